# baseline (speedup 1.0000x reference)
import functools

import jax
import jax.numpy as jnp
from jax import lax
from jax.experimental import pallas as pl
from jax.experimental.pallas import tpu as pltpu
from jax.experimental.pallas import tpu_sc as plsc

_NC = 2
_NS = 16
_LANES = 16
_NW = _NC * _NS


@functools.partial(jax.jit, static_argnums=())
def kernel(x, table, pos_enc):
    B, T = x.shape
    V, D = table.shape
    N = B * T
    scale = float(D) ** 0.5

    rows_per_w = N // _NW
    C = T
    n_chunks = rows_per_w // C

    x_flat = x.reshape(N)

    mesh = plsc.VectorSubcoreMesh(core_axis_name="c", subcore_axis_name="s")

    @functools.partial(
        pl.kernel,
        out_type=jax.ShapeDtypeStruct((N, D), jnp.float32),
        mesh=mesh,
        scratch_types=[
            pltpu.VMEM((C,), jnp.int32),
            pltpu.VMEM((C, D), jnp.float32),
            pltpu.VMEM((T, D), jnp.float32),
            pltpu.SemaphoreType.DMA,
        ],
        compiler_params=pltpu.CompilerParams(use_tc_tiling_on_sc=False),
    )
    def sc_embed(x_hbm, table_hbm, pos_hbm, out_hbm, idx_v, rows_v, pos_v, sem):
        wid = lax.axis_index("s") * _NC + lax.axis_index("c")
        w_base = wid * rows_per_w
        pltpu.sync_copy(pos_hbm, pos_v)

        def chunk_body(g, carry):
            base = w_base + g * C
            pltpu.sync_copy(x_hbm.at[pl.ds(base, C)], idx_v)
            pltpu.async_copy(table_hbm.at[idx_v], rows_v, sem).wait()

            def row_body(r, carry2):
                for c in range(D // _LANES):
                    sl = pl.ds(c * _LANES, _LANES)
                    rows_v[r, sl] = rows_v[r, sl] * scale + pos_v[r, sl]
                return carry2

            lax.fori_loop(0, C, row_body, 0, unroll=2)
            pltpu.sync_copy(rows_v, out_hbm.at[pl.ds(base, C)])
            return carry

        lax.fori_loop(0, n_chunks, chunk_body, 0)

    out = sc_embed(x_flat, table, pos_enc)
    return out.reshape(B, T, D)


# speedup vs baseline: 1.1812x; 1.1812x over previous
import functools

import jax
import jax.numpy as jnp
from jax import lax
from jax.experimental import pallas as pl
from jax.experimental.pallas import tpu as pltpu
from jax.experimental.pallas import tpu_sc as plsc

_NC = 2
_NS = 16
_LANES = 16
_NW = _NC * _NS
_NBUF = 4


@functools.partial(jax.jit, static_argnums=())
def kernel(x, table, pos_enc):
    B, T = x.shape
    V, D = table.shape
    N = B * T
    scale = float(D) ** 0.5

    rows_per_w = N // _NW
    C = T
    n_chunks = rows_per_w // C

    x_w = x.reshape(_NW, n_chunks, C)

    mesh = plsc.VectorSubcoreMesh(core_axis_name="c", subcore_axis_name="s")

    @functools.partial(
        pl.kernel,
        out_type=jax.ShapeDtypeStruct((N, D), jnp.float32),
        mesh=mesh,
        scratch_types=[
            pltpu.VMEM((n_chunks, C), jnp.int32),
            pltpu.VMEM((T, D), jnp.float32),
            pltpu.VMEM((_NBUF, C, D), jnp.float32),
            pltpu.SemaphoreType.DMA((_NBUF,)),
            pltpu.SemaphoreType.DMA((_NBUF,)),
        ],
        compiler_params=pltpu.CompilerParams(use_tc_tiling_on_sc=False),
    )
    def sc_embed(x_hbm, table_hbm, pos_hbm, out_hbm, idx_v, pos_v, rows_v, gsem, osem):
        wid = lax.axis_index("s") * _NC + lax.axis_index("c")
        w_base = wid * rows_per_w
        pltpu.sync_copy(x_hbm.at[wid], idx_v)
        pltpu.sync_copy(pos_hbm, pos_v)

        def start_gather(g, b):
            pltpu.async_copy(table_hbm.at[idx_v.at[g]], rows_v.at[b], gsem.at[b])

        def wait_gather(g, b):
            pltpu.make_async_copy(
                table_hbm.at[idx_v.at[g]], rows_v.at[b], gsem.at[b]
            ).wait()

        def start_out(g, b):
            pltpu.async_copy(
                rows_v.at[b], out_hbm.at[pl.ds(w_base + g * C, C)], osem.at[b]
            )

        def wait_out(g, b):
            pltpu.make_async_copy(
                rows_v.at[b], out_hbm.at[pl.ds(w_base + g * C, C)], osem.at[b]
            ).wait()

        start_gather(0, 0)
        start_gather(1, 1)

        def chunk_body(g, carry):
            b = lax.rem(g, _NBUF)
            wait_gather(g, b)

            def row_body(r, carry2):
                for c in range(D // _LANES):
                    sl = pl.ds(c * _LANES, _LANES)
                    rows_v[b, r, sl] = rows_v[b, r, sl] * scale + pos_v[r, sl]
                return carry2

            lax.fori_loop(0, C, row_body, 0, unroll=2)
            start_out(g, b)

            b2 = lax.rem(g + 2, _NBUF)

            @pl.when(g >= 2)
            def _():
                wait_out(g - 2, b2)

            @pl.when(g + 2 < n_chunks)
            def _():
                start_gather(g + 2, b2)

            return carry

        lax.fori_loop(0, n_chunks, chunk_body, 0)
        wait_out(n_chunks - 2, lax.rem(n_chunks - 2, _NBUF))
        wait_out(n_chunks - 1, lax.rem(n_chunks - 1, _NBUF))

    out = sc_embed(x_w, table, pos_enc)
    return out.reshape(B, T, D)


# speedup vs baseline: 1.4787x; 1.2518x over previous
import functools

import jax
import jax.numpy as jnp
from jax import lax
from jax.experimental import pallas as pl
from jax.experimental.pallas import tpu as pltpu
from jax.experimental.pallas import tpu_sc as plsc

_NC = 2
_NS = 16
_LANES = 16
_NW = _NC * _NS
_NBUF = 4


@functools.partial(jax.jit, static_argnums=())
def kernel(x, table, pos_enc):
    B, T = x.shape
    V, D = table.shape
    N = B * T
    scale = float(D) ** 0.5

    rows_per_w = N // _NW
    C = T
    n_chunks = rows_per_w // C

    x_w = x.reshape(_NW, n_chunks, C)

    mesh = plsc.VectorSubcoreMesh(core_axis_name="c", subcore_axis_name="s")

    @functools.partial(
        pl.kernel,
        out_type=jax.ShapeDtypeStruct((N, D), jnp.float32),
        mesh=mesh,
        scratch_types=[
            pltpu.VMEM((n_chunks, C), jnp.int32),
            pltpu.VMEM((T, D), jnp.float32),
            pltpu.VMEM((_NBUF, C, D), jnp.float32),
            pltpu.SemaphoreType.DMA((_NBUF,)),
            pltpu.SemaphoreType.DMA((_NBUF,)),
        ],
        compiler_params=pltpu.CompilerParams(use_tc_tiling_on_sc=False),
    )
    def sc_embed(x_hbm, table_hbm, pos_hbm, out_hbm, idx_v, pos_v, rows_v, gsem, osem):
        wid = lax.axis_index("s") * _NC + lax.axis_index("c")
        w_base = wid * rows_per_w
        pltpu.sync_copy(x_hbm.at[wid], idx_v)
        pltpu.sync_copy(pos_hbm, pos_v)

        def start_gather(g, b):
            pltpu.async_copy(table_hbm.at[idx_v.at[g]], rows_v.at[b], gsem.at[b])

        def wait_gather(g, b):
            pltpu.make_async_copy(
                table_hbm.at[idx_v.at[g]], rows_v.at[b], gsem.at[b]
            ).wait()

        def start_out(g, b):
            pltpu.async_copy(
                rows_v.at[b], out_hbm.at[pl.ds(w_base + g * C, C)], osem.at[b]
            )

        def wait_out(g, b):
            pltpu.make_async_copy(
                rows_v.at[b], out_hbm.at[pl.ds(w_base + g * C, C)], osem.at[b]
            ).wait()

        start_gather(0, 0)
        start_gather(1, 1)

        def chunk_body(g, carry):
            b = lax.rem(g, _NBUF)
            wait_gather(g, b)

            @plsc.parallel_loop(0, C, unroll=4)
            def _compute(r):
                for c in range(D // _LANES):
                    sl = pl.ds(c * _LANES, _LANES)
                    rows_v[b, r, sl] = rows_v[b, r, sl] * scale + pos_v[r, sl]
            start_out(g, b)

            b2 = lax.rem(g + 2, _NBUF)

            @pl.when(g >= 2)
            def _():
                wait_out(g - 2, b2)

            @pl.when(g + 2 < n_chunks)
            def _():
                start_gather(g + 2, b2)

            return carry

        lax.fori_loop(0, n_chunks, chunk_body, 0)
        wait_out(n_chunks - 2, lax.rem(n_chunks - 2, _NBUF))
        wait_out(n_chunks - 1, lax.rem(n_chunks - 1, _NBUF))

    out = sc_embed(x_w, table, pos_enc)
    return out.reshape(B, T, D)
